# baseline (speedup 1.0000x reference)
import functools

import jax
import jax.numpy as jnp
from jax import lax
from jax.experimental import pallas as pl
from jax.experimental.pallas import tpu as pltpu
from jax.experimental.pallas import tpu_sc as plsc

D = 32
NB = 16384
NJ = 50
B = NB * NJ
NC = 2
NS = 16
NW = NC * NS
IW = 128
CHUNK = 512
PER_W = B // NW
NCHUNK = PER_W // CHUNK
NBUF = 2

_mesh = plsc.VectorSubcoreMesh(core_axis_name="c", subcore_axis_name="s")


@functools.partial(
    pl.kernel,
    out_type=jax.ShapeDtypeStruct((B // CHUNK, IW, IW // D, D), jnp.float32),
    mesh=_mesh,
    scratch_types=[
        pltpu.VMEM((PER_W,), jnp.int32),
        pltpu.VMEM((NBUF, CHUNK, D), jnp.float32),
        pltpu.SemaphoreType.DMA,
        pltpu.SemaphoreType.DMA,
    ],
    compiler_params=pltpu.CompilerParams(use_tc_tiling_on_sc=False),
)
def _sc_gather(idx_hbm, table_hbm, out_hbm, idx_v, rows_v, gsem, osem):
    wid = lax.axis_index("s") * NC + lax.axis_index("c")
    base = wid * PER_W
    pltpu.sync_copy(
        idx_hbm.at[pl.ds(pl.multiple_of(base, PER_W), PER_W)], idx_v
    )

    def fire(g):
        b = g % NBUF
        return [
            pltpu.async_copy(
                table_hbm.at[idx_v.at[pl.ds(g * CHUNK + p * IW, IW)]],
                rows_v.at[b].at[pl.ds(p * IW, IW)],
                gsem,
            )
            for p in range(CHUNK // IW)
        ]

    def start_out(g):
        b = g % NBUF
        gc = pl.multiple_of(wid * NCHUNK + g, 1)
        return [
            pltpu.async_copy(
                rows_v.at[b].at[pl.ds(p * IW, IW)],
                out_hbm.at[gc].at[:, p],
                osem,
            )
            for p in range(CHUNK // IW)
        ]

    gathers = {g: fire(g) for g in range(min(NBUF, NCHUNK))}
    outs = {}
    for g in range(NCHUNK):
        for c in gathers.pop(g):
            c.wait()
        outs[g] = start_out(g)
        nxt = g + NBUF
        if nxt < NCHUNK:
            for c in outs.pop(nxt - NBUF):
                c.wait()
            gathers[nxt] = fire(nxt)
    for cs in outs.values():
        for c in cs:
            c.wait()


NT = 500000
TBLK = 2048


def _tc_tprep_body(in_ref, out_ref):
    y = in_ref[...]
    for k in range(TBLK // IW):
        out_ref[k * IW:(k + 1) * IW, :] = y[:, k * IW:(k + 1) * IW].T


_tc_tprep = pl.pallas_call(
    _tc_tprep_body,
    grid=((NT + TBLK - 1) // TBLK,),
    in_specs=[pl.BlockSpec((D, TBLK), lambda i: (0, i))],
    out_specs=pl.BlockSpec((TBLK, D), lambda i: (i, 0)),
    out_shape=jax.ShapeDtypeStruct((NT, D), jnp.float32),
)


PK = IW // D
NTT = 2
PR = NB * D // IW
CPB = PR // NTT // IW


def _tc_transpose_body(in_ref, out_ref):
    y = in_ref[...]
    for t in range(CPB):
        for p in range(PK):
            out_ref[0, :, t * CHUNK + p * IW:t * CHUNK + (p + 1) * IW] = (
                y[t * IW:(t + 1) * IW, p * D:(p + 1) * D].T
            )


_tc_transpose = pl.pallas_call(
    _tc_transpose_body,
    grid=(NJ, NTT),
    in_specs=[
        pl.BlockSpec((PR // NTT, IW), lambda j, t: (j * NTT + t, 0)),
    ],
    out_specs=pl.BlockSpec((1, D, NB // NTT), lambda j, t: (j, 0, t)),
    out_shape=jax.ShapeDtypeStruct((NJ, D, NB), jnp.float32),
)


def kernel(nodes, W1):
    idx = jnp.transpose(nodes).reshape(B).astype(jnp.int32)
    flat = _sc_gather(idx, _tc_tprep(jnp.transpose(W1)))
    out2 = _tc_transpose(flat.reshape(B * D // IW, IW))
    return jnp.transpose(out2, (2, 0, 1))


# speedup vs baseline: 1.2525x; 1.2525x over previous
import functools

import jax
import jax.numpy as jnp
from jax import lax
from jax.experimental import pallas as pl
from jax.experimental.pallas import tpu as pltpu
from jax.experimental.pallas import tpu_sc as plsc

D = 32
NB = 16384
NJ = 50
B = NB * NJ
NC = 2
NS = 16
NW = NC * NS
IW = 128
CHUNK = 512
PER_W = B // NW
NCHUNK = PER_W // CHUNK
NBUF = 2

_mesh = plsc.VectorSubcoreMesh(core_axis_name="c", subcore_axis_name="s")


@functools.partial(
    pl.kernel,
    out_type=jax.ShapeDtypeStruct((B // CHUNK, IW, IW // D, D), jnp.float32),
    mesh=_mesh,
    scratch_types=[
        pltpu.VMEM((PER_W,), jnp.int32),
        pltpu.VMEM((NBUF, CHUNK, D), jnp.float32),
        pltpu.SemaphoreType.DMA,
        pltpu.SemaphoreType.DMA,
    ],
    compiler_params=pltpu.CompilerParams(use_tc_tiling_on_sc=False),
)
def _sc_gather(idx_hbm, table_hbm, out_hbm, idx_v, rows_v, gsem, osem):
    wid = lax.axis_index("s") * NC + lax.axis_index("c")
    base = wid * PER_W
    pltpu.sync_copy(
        idx_hbm.at[pl.ds(pl.multiple_of(base, PER_W), PER_W)], idx_v
    )

    def fire(g):
        b = g % NBUF
        return [
            pltpu.async_copy(
                table_hbm.at[idx_v.at[pl.ds(g * CHUNK + p * IW, IW)]],
                rows_v.at[b].at[pl.ds(p * IW, IW)],
                gsem,
            )
            for p in range(CHUNK // IW)
        ]

    def start_out(g):
        b = g % NBUF
        gc = pl.multiple_of(wid * NCHUNK + g, 1)
        return [
            pltpu.async_copy(
                rows_v.at[b].at[pl.ds(p * IW, IW)],
                out_hbm.at[gc].at[:, p],
                osem,
            )
            for p in range(CHUNK // IW)
        ]

    gathers = {g: fire(g) for g in range(min(NBUF, NCHUNK))}
    outs = {}
    for g in range(NCHUNK):
        for c in gathers.pop(g):
            c.wait()
        outs[g] = start_out(g)
        nxt = g + NBUF
        if nxt < NCHUNK:
            for c in outs.pop(nxt - NBUF):
                c.wait()
            gathers[nxt] = fire(nxt)
    for cs in outs.values():
        for c in cs:
            c.wait()


PK = IW // D
NTT = 2
PR = NB * D // IW
CPB = PR // NTT // IW


def _tc_transpose_body(in_ref, out_ref):
    y = in_ref[...]
    for t in range(CPB):
        for p in range(PK):
            out_ref[0, :, t * CHUNK + p * IW:t * CHUNK + (p + 1) * IW] = (
                y[t * IW:(t + 1) * IW, p * D:(p + 1) * D].T
            )


_tc_transpose = pl.pallas_call(
    _tc_transpose_body,
    grid=(NJ, NTT),
    in_specs=[
        pl.BlockSpec((PR // NTT, IW), lambda j, t: (j * NTT + t, 0)),
    ],
    out_specs=pl.BlockSpec((1, D, NB // NTT), lambda j, t: (j, 0, t)),
    out_shape=jax.ShapeDtypeStruct((NJ, D, NB), jnp.float32),
)


def kernel(nodes, W1):
    idx = jnp.transpose(nodes).reshape(B).astype(jnp.int32)
    flat = _sc_gather(idx, W1)
    out2 = _tc_transpose(flat.reshape(B * D // IW, IW))
    return jnp.transpose(out2, (2, 0, 1))
